# baseline (speedup 1.0000x reference)
import functools
import math

import jax
import jax.numpy as jnp
from jax import lax
from jax.experimental import pallas as pl
from jax.experimental.pallas import tpu as pltpu
from jax.experimental.pallas import tpu_sc as plsc

D_MODEL = 128
SCALE = math.sqrt(D_MODEL)
NBUF = 5
PREF = 3


@functools.lru_cache(maxsize=None)
def _make_kernel(seq_len, n_seq):
    info = plsc.get_sparse_core_info()
    nw = info.num_cores * info.num_subcores
    assert n_seq % (nw * 8) == 0
    sw = n_seq // nw
    n_chunks = seq_len
    n_outer = n_chunks // NBUF
    assert n_chunks % NBUF == 0 and n_outer >= 2 and sw <= 128
    mesh = plsc.VectorSubcoreMesh(core_axis_name="c", subcore_axis_name="s")

    @functools.partial(
        pl.kernel,
        mesh=mesh,
        out_type=jax.ShapeDtypeStruct((seq_len, n_seq, D_MODEL), jnp.float32),
        scratch_types=(
            [pltpu.VMEM((n_chunks, sw), jnp.int32)]
            + [pltpu.VMEM((sw, D_MODEL), jnp.float32) for _ in range(NBUF)]
            + [pltpu.SemaphoreType.DMA for _ in range(2 * NBUF)]
        ),
    )
    def emb(lut_hbm, idx_hbm, out_hbm, idx_v, *bufs_sems):
        bufs = bufs_sems[:NBUF]
        gsem = bufs_sems[NBUF:2 * NBUF]
        ssem = bufs_sems[2 * NBUF:]
        wid = lax.axis_index("s") * info.num_cores + lax.axis_index("c")
        base = wid * sw
        pltpu.sync_copy(idx_hbm.at[:, pl.ds(base, sw)], idx_v)

        def gather_start(c, b):
            pltpu.async_copy(lut_hbm.at[idx_v.at[c]], bufs[b], gsem[b])

        def gather_wait(b):
            pltpu.make_async_copy(
                lut_hbm.at[idx_v.at[0]], bufs[b], gsem[b]).wait()

        def store_start(c, b):
            pltpu.async_copy(
                bufs[b], out_hbm.at[c, pl.ds(base, sw)], ssem[b])

        def store_wait(b):
            pltpu.make_async_copy(
                bufs[b], out_hbm.at[0, pl.ds(base, sw)], ssem[b]).wait()

        for b in range(PREF):
            gather_start(b, b)

        def iter_body(j, carry):
            for b in range(NBUF):
                c = j * NBUF + b
                tb = (b + PREF) % NBUF
                if b < NBUF - PREF:
                    @pl.when(j >= 1)
                    def _():
                        store_wait(tb)
                    gather_start(c + PREF, tb)
                else:
                    @pl.when(j <= n_outer - 2)
                    def _():
                        store_wait(tb)
                        gather_start(c + PREF, tb)
                gather_wait(b)
                buf = bufs[b]

                @plsc.parallel_loop(0, sw, unroll=4)
                def _(r):
                    for g in range(D_MODEL // 16):
                        sl = pl.ds(g * 16, 16)
                        buf[r, sl] = buf[r, sl] * SCALE

                store_start(c, b)
            return carry

        lax.fori_loop(0, n_outer, iter_body, 0)
        for b in range(NBUF):
            store_wait(b)

    return emb


@jax.jit
def kernel(x, lut):
    n_seq, seq_len = x.shape
    xt = x.T.astype(jnp.int32)
    out_t = _make_kernel(seq_len, n_seq)(lut, xt)
    return jnp.transpose(out_t, (1, 0, 2))


# speedup vs baseline: 1.0026x; 1.0026x over previous
import functools
import math

import jax
import jax.numpy as jnp
from jax import lax
from jax.experimental import pallas as pl
from jax.experimental.pallas import tpu as pltpu
from jax.experimental.pallas import tpu_sc as plsc

D_MODEL = 128
SCALE = math.sqrt(D_MODEL)
NBUF = 5
PREF = 4


@functools.lru_cache(maxsize=None)
def _make_kernel(seq_len, n_seq):
    info = plsc.get_sparse_core_info()
    nw = info.num_cores * info.num_subcores
    assert n_seq % (nw * 8) == 0
    sw = n_seq // nw
    n_chunks = seq_len
    n_outer = n_chunks // NBUF
    assert n_chunks % NBUF == 0 and n_outer >= 2 and sw <= 128
    mesh = plsc.VectorSubcoreMesh(core_axis_name="c", subcore_axis_name="s")

    @functools.partial(
        pl.kernel,
        mesh=mesh,
        out_type=jax.ShapeDtypeStruct((seq_len, n_seq, D_MODEL), jnp.float32),
        scratch_types=(
            [pltpu.VMEM((n_chunks, sw), jnp.int32)]
            + [pltpu.VMEM((sw, D_MODEL), jnp.float32) for _ in range(NBUF)]
            + [pltpu.SemaphoreType.DMA for _ in range(2 * NBUF)]
        ),
    )
    def emb(lut_hbm, idx_hbm, out_hbm, idx_v, *bufs_sems):
        bufs = bufs_sems[:NBUF]
        gsem = bufs_sems[NBUF:2 * NBUF]
        ssem = bufs_sems[2 * NBUF:]
        wid = lax.axis_index("s") * info.num_cores + lax.axis_index("c")
        base = wid * sw
        pltpu.sync_copy(idx_hbm.at[:, pl.ds(base, sw)], idx_v)

        def gather_start(c, b):
            pltpu.async_copy(lut_hbm.at[idx_v.at[c]], bufs[b], gsem[b])

        def gather_wait(b):
            pltpu.make_async_copy(
                lut_hbm.at[idx_v.at[0]], bufs[b], gsem[b]).wait()

        def store_start(c, b):
            pltpu.async_copy(
                bufs[b], out_hbm.at[c, pl.ds(base, sw)], ssem[b])

        def store_wait(b):
            pltpu.make_async_copy(
                bufs[b], out_hbm.at[0, pl.ds(base, sw)], ssem[b]).wait()

        for b in range(PREF):
            gather_start(b, b)

        def iter_body(j, carry):
            for b in range(NBUF):
                c = j * NBUF + b
                tb = (b + PREF) % NBUF
                if b < NBUF - PREF:
                    @pl.when(j >= 1)
                    def _():
                        store_wait(tb)
                    gather_start(c + PREF, tb)
                else:
                    @pl.when(j <= n_outer - 2)
                    def _():
                        store_wait(tb)
                        gather_start(c + PREF, tb)
                gather_wait(b)
                buf = bufs[b]

                @plsc.parallel_loop(0, sw, unroll=2)
                def _(r):
                    for g in range(D_MODEL // 16):
                        sl = pl.ds(g * 16, 16)
                        buf[r, sl] = buf[r, sl] * SCALE

                store_start(c, b)
            return carry

        lax.fori_loop(0, n_outer, iter_body, 0)
        for b in range(NBUF):
            store_wait(b)

    return emb


@jax.jit
def kernel(x, lut):
    n_seq, seq_len = x.shape
    xt = x.T.astype(jnp.int32)
    out_t = _make_kernel(seq_len, n_seq)(lut, xt)
    return jnp.transpose(out_t, (1, 0, 2))
